# Initial kernel scaffold; baseline (speedup 1.0000x reference)
import jax
import jax.numpy as jnp
from jax.experimental import pallas as pl


def kernel(X, M_0, W_k, b_k, W_v, b_v, W_g, b_g, W_q, b_q):
    raise NotImplementedError("write your pallas kernel here")


# speedup vs baseline: 20.0471x; 20.0471x over previous
import jax
import jax.numpy as jnp
from jax.experimental import pallas as pl

_T = 16
_B = 32
_DIN = 1024
_D = 64
_N = 32
_K = 4
_NC = _N + 1
_DKV = _D * _NC
_SN = _T * _NC
_R = _B * _T
_G = 4
_GSN = _G * _SN
_GT = _G * _T
_KCHUNK = 256


def _proj_kv_kernel(x_ref, wk_ref, bk_ref, wv_ref, bv_ref, yk_ref, yv_ref):
    j = pl.program_id(0)
    x = x_ref[...]
    dims = (((1,), (1,)), ((), ()))
    pk = jax.lax.dot_general(x, wk_ref[...], dims, preferred_element_type=jnp.float32)
    pv = jax.lax.dot_general(x, wv_ref[...], dims, preferred_element_type=jnp.float32)

    @pl.when(j == 0)
    def _():
        yk_ref[...] = pk + bk_ref[...]
        yv_ref[...] = pv + bv_ref[...]

    @pl.when(j != 0)
    def _():
        yk_ref[...] += pk
        yv_ref[...] += pv


def _proj_gq_kernel(x_ref, wg_ref, bg_ref, wq_ref, bq_ref, yg_ref, yq_ref):
    x = x_ref[...]
    dims = (((1,), (1,)), ((), ()))
    yg_ref[...] = (
        jax.lax.dot_general(x, wg_ref[...], dims, preferred_element_type=jnp.float32)
        + bg_ref[...]
    )
    yq_ref[...] = (
        jax.lax.dot_general(x, wq_ref[...], dims, preferred_element_type=jnp.float32)
        + bq_ref[...]
    )


def _route_kernel(g_ref, w_ref, sel_ref):
    g = g_ref[...]
    mx = jnp.max(g, axis=1, keepdims=True)
    e = jnp.exp(g - mx)
    sig = e / jnp.sum(e, axis=1, keepdims=True)

    iota_n = jax.lax.broadcasted_iota(jnp.int32, (_R, _N), 1)
    work = sig
    wn = jnp.zeros((_R, _N), jnp.float32)
    seln = jnp.zeros((_R, _N), jnp.float32)
    den = jnp.zeros((_R, 1), jnp.float32)
    for _ in range(_K):
        mj = jnp.max(work, axis=1, keepdims=True)
        eq = work == mj
        first = jnp.min(jnp.where(eq, iota_n, _N), axis=1, keepdims=True)
        oh = iota_n == first
        wn = wn + jnp.where(oh, work, 0.0)
        seln = jnp.maximum(seln, oh.astype(jnp.float32))
        den = den + mj
        work = jnp.where(oh, -1.0, work)
    wn = wn / den

    ones_col = jnp.ones((_R, 1), jnp.float32)
    w_ref[...] = jnp.concatenate([ones_col, wn], axis=1)
    sel_ref[...] = jnp.concatenate([ones_col, seln], axis=1)


def _attn_kernel(w_ref, selflat_ref, q_ref, k_ref, v_ref, m0_ref, o_ref):
    w33 = w_ref[...].reshape(_GT, _NC)
    selflat = selflat_ref[0]
    q = q_ref[...].reshape(_GT, _D)
    kf = k_ref[...].reshape(_GSN, _D)
    vf = v_ref[...].reshape(_GSN, _D)
    m0 = m0_ref[...]

    iota_r_nc = jax.lax.broadcasted_iota(jnp.int32, (_NC, _SN), 0)
    iota_c_nc = jax.lax.broadcasted_iota(jnp.int32, (_NC, _SN), 1)
    e_n = (iota_c_nc % _NC == iota_r_nc).astype(jnp.float32)
    wexp_small = jnp.dot(w33, e_n, preferred_element_type=jnp.float32)
    wexp = jnp.concatenate([wexp_small] * _G, axis=1)

    iota_r = jax.lax.broadcasted_iota(jnp.int32, (_GT, _GSN), 0)
    iota_c = jax.lax.broadcasted_iota(jnp.int32, (_GT, _GSN), 1)
    gb_r = iota_r // _T
    gb_c = iota_c // _SN
    t_r = iota_r - gb_r * _T
    s_c = (iota_c - gb_c * _SN) // _NC
    keep = jnp.logical_and(gb_r == gb_c, t_r >= s_c)

    s_mat = jax.lax.dot_general(
        q, kf, (((1,), (1,)), ((), ())), preferred_element_type=jnp.float32
    )
    p = jnp.where(keep, s_mat * wexp * selflat, 0.0)
    o1 = jnp.dot(p, vf, preferred_element_type=jnp.float32)
    o_ref[...] = (
        o1 + 2.0 * jnp.dot(q, m0, preferred_element_type=jnp.float32)
    ).reshape(1, _G, _T, _D)


@jax.jit
def kernel(X, M_0, W_k, b_k, W_v, b_v, W_g, b_g, W_q, b_q):
    xb = X.transpose(1, 0, 2).reshape(_R, _DIN)

    yk, yv = pl.pallas_call(
        _proj_kv_kernel,
        grid=(_DIN // _KCHUNK,),
        in_specs=[
            pl.BlockSpec((_R, _KCHUNK), lambda j: (0, j)),
            pl.BlockSpec((_DKV, _KCHUNK), lambda j: (0, j)),
            pl.BlockSpec((1, _DKV), lambda j: (0, 0)),
            pl.BlockSpec((_DKV, _KCHUNK), lambda j: (0, j)),
            pl.BlockSpec((1, _DKV), lambda j: (0, 0)),
        ],
        out_specs=(
            pl.BlockSpec((_R, _DKV), lambda j: (0, 0)),
            pl.BlockSpec((_R, _DKV), lambda j: (0, 0)),
        ),
        out_shape=(
            jax.ShapeDtypeStruct((_R, _DKV), jnp.float32),
            jax.ShapeDtypeStruct((_R, _DKV), jnp.float32),
        ),
    )(xb, W_k, b_k[None, :], W_v, b_v[None, :])

    yg, yq = pl.pallas_call(
        _proj_gq_kernel,
        out_shape=(
            jax.ShapeDtypeStruct((_R, _N), jnp.float32),
            jax.ShapeDtypeStruct((_R, _D), jnp.float32),
        ),
    )(xb, W_g, b_g[None, :], W_q, b_q[None, :])

    w33, sel33 = pl.pallas_call(
        _route_kernel,
        out_shape=(
            jax.ShapeDtypeStruct((_R, _NC), jnp.float32),
            jax.ShapeDtypeStruct((_R, _NC), jnp.float32),
        ),
    )(yg)

    nblk = _B // _G
    w33b = w33.reshape(nblk, _G, _T, _NC)
    selflat = sel33.reshape(nblk, 1, _GSN)
    qb = yq.reshape(nblk, _G, _T, _D)
    kf = yk.reshape(nblk, _G, _SN, _D)
    vf = yv.reshape(nblk, _G, _SN, _D)

    o = pl.pallas_call(
        _attn_kernel,
        grid=(nblk,),
        in_specs=[
            pl.BlockSpec((1, _G, _T, _NC), lambda i: (i, 0, 0, 0)),
            pl.BlockSpec((1, 1, _GSN), lambda i: (i, 0, 0)),
            pl.BlockSpec((1, _G, _T, _D), lambda i: (i, 0, 0, 0)),
            pl.BlockSpec((1, _G, _SN, _D), lambda i: (i, 0, 0, 0)),
            pl.BlockSpec((1, _G, _SN, _D), lambda i: (i, 0, 0, 0)),
            pl.BlockSpec((_D, _D), lambda i: (0, 0)),
        ],
        out_specs=pl.BlockSpec((1, _G, _T, _D), lambda i: (i, 0, 0, 0)),
        out_shape=jax.ShapeDtypeStruct((nblk, _G, _T, _D), jnp.float32),
    )(w33b, selflat, qb, kf, vf, M_0)

    return o.reshape(_B, _T, _D).transpose(1, 0, 2)
